# Initial kernel scaffold; baseline (speedup 1.0000x reference)
import jax
import jax.numpy as jnp
from jax.experimental import pallas as pl


def kernel(x, edge_index, batch, c1_W1, c1_b1, c1_W2, c1_b2, c1_g, c1_bt, c2_W1, c2_b1, c2_W2, c2_b2, c2_g, c2_bt, f1_W, f1_b, f1_g, f1_bt, f2_W, f2_b, f2_g, f2_bt, m_W1, m_b1, m_g, m_bt, m_a, m_W2, m_b2):
    raise NotImplementedError("write your pallas kernel here")


# speedup vs baseline: 4.7077x; 4.7077x over previous
import functools

import jax
import jax.numpy as jnp
from jax import lax
from jax.experimental import pallas as pl
from jax.experimental.pallas import tpu as pltpu
from jax.experimental.pallas import tpu_sc as plsc

_G = 64
_RS = (1.0 + 1e-5) ** -0.5
_PREC = lax.Precision.DEFAULT


@functools.lru_cache(maxsize=None)
def _make_agg(N, D, E):
    info = plsc.get_sparse_core_info()
    NC, NS = info.num_cores, info.num_subcores
    NW = NC * NS
    assert E % NW == 0
    EPW = E // NW
    CH = 80
    assert EPW % CH == 0 and CH % 8 == 0
    NCH = EPW // CH
    W_CHK = (N // NS) // 8 * 8
    TAIL = N - NS * W_CHK
    ZR = 78
    assert W_CHK % ZR == 0 and TAIL % 8 == 0 and TAIL <= ZR
    mesh = plsc.VectorSubcoreMesh(core_axis_name="c", subcore_axis_name="s")

    @functools.partial(
        pl.kernel,
        out_type=jax.ShapeDtypeStruct((NC, N, D), jnp.float32),
        mesh=mesh,
        scratch_types=[
            pltpu.VMEM((CH,), jnp.int32),
            pltpu.VMEM((CH,), jnp.int32),
            pltpu.VMEM((CH, D), jnp.float32),
            pltpu.VMEM((ZR, D), jnp.float32),
            pltpu.VMEM_SHARED((N, D), jnp.float32),
            pltpu.SemaphoreType.DMA,
        ],
    )
    def agg(x_hbm, src_hbm, dst_hbm, out_hbm, src_v, dst_v, rows_v, zbuf, acc, sem):
        cid = lax.axis_index("c")
        sid = lax.axis_index("s")
        wid = sid * NC + cid
        def zrow(i, carry):
            for j in range(D // 16):
                zbuf[i, pl.ds(j * 16, 16)] = jnp.zeros((16,), jnp.float32)
            return carry
        lax.fori_loop(0, ZR, zrow, 0)
        row0 = sid * W_CHK
        for k in range(W_CHK // ZR):
            pltpu.sync_copy(zbuf, acc.at[pl.ds(row0 + k * ZR, ZR)])

        @pl.when(sid == NS - 1)
        def _():
            pltpu.sync_copy(zbuf.at[pl.ds(0, TAIL)], acc.at[pl.ds(NS * W_CHK, TAIL)])

        plsc.subcore_barrier()
        base = wid * EPW
        def body(i, carry):
            off = base + i * CH
            pltpu.sync_copy(src_hbm.at[pl.ds(off, CH)], src_v)
            pltpu.sync_copy(dst_hbm.at[pl.ds(off, CH)], dst_v)
            pltpu.async_copy(x_hbm.at[src_v], rows_v, sem).wait()
            pltpu.sync_copy(rows_v, acc.at[dst_v], add=True)
            return carry
        lax.fori_loop(0, NCH, body, 0)
        plsc.subcore_barrier()
        pltpu.sync_copy(acc.at[pl.ds(row0, W_CHK)], out_hbm.at[cid, pl.ds(row0, W_CHK)])

        @pl.when(sid == NS - 1)
        def _():
            pltpu.sync_copy(acc.at[pl.ds(NS * W_CHK, TAIL)],
                            out_hbm.at[cid, pl.ds(NS * W_CHK, TAIL)])

    return agg


@functools.lru_cache(maxsize=None)
def _make_gin(N, D, H, G, BLK):
    NB = N // BLK

    def body(x_ref, p0_ref, p1_ref, bat_ref, W1_ref, b1_ref, W2_ref, b2_ref,
             g_ref, bt_ref, y_ref, px_ref, py_ref):
        i = pl.program_id(0)
        xb = x_ref[...]
        z = xb + p0_ref[...] + p1_ref[...]
        h = jnp.dot(z, W1_ref[...], preferred_element_type=jnp.float32,
                    precision=_PREC) + b1_ref[...]
        h = jnp.maximum(h, 0.0)
        h = jnp.dot(h, W2_ref[...], preferred_element_type=jnp.float32,
                    precision=_PREC) + b2_ref[...]
        h = jnp.maximum(h, 0.0)
        y = h * (g_ref[...] * _RS) + bt_ref[...]
        y_ref[...] = y
        iota = lax.broadcasted_iota(jnp.int32, (G, BLK), 0)
        m = (bat_ref[0] == iota).astype(jnp.float32)

        @pl.when(i == 0)
        def _():
            px_ref[...] = jnp.zeros_like(px_ref)
            py_ref[...] = jnp.zeros_like(py_ref)

        px_ref[...] += jnp.dot(m, xb, preferred_element_type=jnp.float32,
                               precision=_PREC)
        py_ref[...] += jnp.dot(m, y, preferred_element_type=jnp.float32,
                               precision=_PREC)

    return pl.pallas_call(
        body,
        grid=(NB,),
        in_specs=[
            pl.BlockSpec((BLK, D), lambda i: (i, 0)),
            pl.BlockSpec((BLK, D), lambda i: (i, 0)),
            pl.BlockSpec((BLK, D), lambda i: (i, 0)),
            pl.BlockSpec((1, 1, BLK), lambda i: (i, 0, 0)),
            pl.BlockSpec((D, H), lambda i: (0, 0)),
            pl.BlockSpec((1, H), lambda i: (0, 0)),
            pl.BlockSpec((H, H), lambda i: (0, 0)),
            pl.BlockSpec((1, H), lambda i: (0, 0)),
            pl.BlockSpec((1, H), lambda i: (0, 0)),
            pl.BlockSpec((1, H), lambda i: (0, 0)),
        ],
        out_specs=[
            pl.BlockSpec((BLK, H), lambda i: (i, 0)),
            pl.BlockSpec((G, D), lambda i: (0, 0)),
            pl.BlockSpec((G, H), lambda i: (0, 0)),
        ],
        out_shape=[
            jax.ShapeDtypeStruct((N, H), jnp.float32),
            jax.ShapeDtypeStruct((G, D), jnp.float32),
            jax.ShapeDtypeStruct((G, H), jnp.float32),
        ],
    )


def _head_body(px_ref, p1_ref, p2_ref, f1W, f1b, f1g, f1bt, f2W, f2b, f2g,
               f2bt, mW1, mb1, mg, mbt, ma, mW2, mb2, o_ref):
    def fc(v, W, b, g, bt):
        h = jnp.dot(v, W[...], preferred_element_type=jnp.float32,
                    precision=_PREC) + b[...]
        h = jnp.maximum(h, 0.0)
        return h * (g[...] * _RS) + bt[...]

    x0 = fc(px_ref[...], f1W, f1b, f1g, f1bt)
    x1g = fc(x0 + p1_ref[...], f2W, f2b, f2g, f2bt)
    x2g = fc(x0 + x1g + p2_ref[...], f2W, f2b, f2g, f2bt)
    h = jnp.dot(x2g, mW1[...], preferred_element_type=jnp.float32,
                precision=_PREC) + mb1[...]
    h = h * (mg[...] * _RS) + mbt[...]
    h = jnp.where(h >= 0.0, h, ma[...] * h)
    o_ref[...] = jnp.dot(h, mW2[...], preferred_element_type=jnp.float32,
                         precision=_PREC) + mb2[...]


@functools.lru_cache(maxsize=None)
def _make_head(G, OUT):
    return pl.pallas_call(
        _head_body,
        out_shape=jax.ShapeDtypeStruct((G, OUT), jnp.float32),
    )


def kernel(x, edge_index, batch, c1_W1, c1_b1, c1_W2, c1_b2, c1_g, c1_bt,
           c2_W1, c2_b1, c2_W2, c2_b2, c2_g, c2_bt, f1_W, f1_b, f1_g, f1_bt,
           f2_W, f2_b, f2_g, f2_bt, m_W1, m_b1, m_g, m_bt, m_a, m_W2, m_b2):
    N, D = x.shape
    H = c1_W2.shape[0]
    E = edge_index.shape[1]
    OUT = m_W2.shape[1]
    G = _G
    BLK = 1000
    src = edge_index[0]
    dst = edge_index[1]
    bat3 = batch.reshape(N // BLK, 1, BLK)
    r2 = lambda v: v.reshape(1, -1)

    agg = _make_agg(N, D, E)
    gin = _make_gin(N, D, H, G, BLK)
    head = _make_head(G, OUT)

    p = agg(x, src, dst)
    x1, pool_x, pool_x1 = gin(x, p[0], p[1], bat3, c1_W1, r2(c1_b1), c1_W2,
                              r2(c1_b2), r2(c1_g), r2(c1_bt))
    q = agg(x1, src, dst)
    _, _, pool_x2 = gin(x1, q[0], q[1], bat3, c2_W1, r2(c2_b1), c2_W2,
                        r2(c2_b2), r2(c2_g), r2(c2_bt))
    return head(pool_x, pool_x1, pool_x2, f1_W, r2(f1_b), r2(f1_g), r2(f1_bt),
                f2_W, r2(f2_b), r2(f2_g), r2(f2_bt), m_W1, r2(m_b1), r2(m_g),
                r2(m_bt), m_a.reshape(1, 1), m_W2, r2(m_b2))
